# baseline (speedup 1.0000x reference)
import functools

import jax
import jax.numpy as jnp
from jax import lax
from jax.experimental import pallas as pl
from jax.experimental.pallas import tpu as pltpu
from jax.experimental.pallas import tpu_sc as plsc

_D = 128
_NC = 2
_NS = 16
_NW = _NC * _NS
_C = 128
_NB = 6


@jax.jit
def _flat_gather(idx, table):
    n_rows = idx.shape[0]
    b_per_w = n_rows // _NW
    nch = b_per_w // _C
    mesh = plsc.VectorSubcoreMesh(core_axis_name="c", subcore_axis_name="s")

    @functools.partial(
        pl.kernel,
        out_type=jax.ShapeDtypeStruct((n_rows, _D), jnp.float32),
        mesh=mesh,
        scratch_types=[
            pltpu.VMEM((b_per_w,), jnp.int32),
            pltpu.VMEM((_NB, _C, _D), jnp.float32),
            pltpu.SemaphoreType.DMA,
            pltpu.SemaphoreType.DMA,
        ],
    )
    def run(idx_hbm, table_hbm, out_hbm, idx_v, rows_v, gsem, osem):
        wid = lax.axis_index("s") * _NC + lax.axis_index("c")
        base = wid * b_per_w
        pltpu.sync_copy(idx_hbm.at[pl.ds(base, b_per_w)], idx_v)

        def g_copy(ch, b):
            return pltpu.make_async_copy(
                table_hbm.at[idx_v.at[pl.ds(ch * _C, _C)]], rows_v.at[b], gsem
            )

        def o_copy(ch, b):
            return pltpu.make_async_copy(
                rows_v.at[b], out_hbm.at[pl.ds(base + ch * _C, _C)], osem
            )

        g_copy(0, 0).start()
        g_copy(1, 1).start()

        @pl.loop(0, nch)
        def chunk_loop(ch):
            b = lax.rem(ch, _NB)
            g_copy(ch, b).wait()
            o_copy(ch, b).start()

            @pl.when(ch + 2 < nch)
            def _():
                @pl.when(ch >= _NB - 2)
                def _():
                    o_copy(ch - _NB + 2, lax.rem(ch + 2, _NB)).wait()

                g_copy(ch + 2, lax.rem(ch + 2, _NB)).start()

        @pl.loop(nch - _NB, nch)
        def drain(k):
            o_copy(k, lax.rem(k, _NB)).wait()

    return run(idx, table)


def kernel(x, embedding):
    n_samp, width = x.shape
    idx = x.T.reshape(n_samp * width).astype(jnp.int32)
    out = _flat_gather(idx, embedding)
    return out.reshape(width, n_samp, _D).swapaxes(0, 1)


# speedup vs baseline: 1.0138x; 1.0138x over previous
import functools

import jax
import jax.numpy as jnp
from jax import lax
from jax.experimental import pallas as pl
from jax.experimental.pallas import tpu as pltpu
from jax.experimental.pallas import tpu_sc as plsc

_D = 128
_NC = 2
_NS = 16
_NW = _NC * _NS
_NB = 6


@jax.jit
def _flat_gather(idx_t, table):
    nch, n_samp = idx_t.shape
    cb = n_samp // _NW
    mesh = plsc.VectorSubcoreMesh(core_axis_name="c", subcore_axis_name="s")

    @functools.partial(
        pl.kernel,
        out_type=jax.ShapeDtypeStruct((nch * n_samp, _D), jnp.float32),
        mesh=mesh,
        scratch_types=[
            pltpu.VMEM((nch, cb), jnp.int32),
            pltpu.VMEM((_NB, cb, _D), jnp.float32),
            pltpu.SemaphoreType.DMA,
            pltpu.SemaphoreType.DMA,
        ],
    )
    def run(idx_hbm, table_hbm, out_hbm, idx_v, rows_v, gsem, osem):
        wid = lax.axis_index("s") * _NC + lax.axis_index("c")
        col0 = wid * cb
        pltpu.sync_copy(idx_hbm.at[:, pl.ds(col0, cb)], idx_v)

        def g_copy(ch, b):
            return pltpu.make_async_copy(
                table_hbm.at[idx_v.at[ch]], rows_v.at[b], gsem
            )

        def o_copy(ch, b):
            return pltpu.make_async_copy(
                rows_v.at[b], out_hbm.at[pl.ds(ch * n_samp + col0, cb)], osem
            )

        g_copy(0, 0).start()
        g_copy(1, 1).start()

        @pl.loop(0, nch)
        def chunk_loop(ch):
            b = lax.rem(ch, _NB)
            g_copy(ch, b).wait()
            o_copy(ch, b).start()

            @pl.when(ch + 2 < nch)
            def _():
                @pl.when(ch >= _NB - 2)
                def _():
                    o_copy(ch - _NB + 2, lax.rem(ch + 2, _NB)).wait()

                g_copy(ch + 2, lax.rem(ch + 2, _NB)).start()

        @pl.loop(nch - _NB, nch)
        def drain(k):
            o_copy(k, lax.rem(k, _NB)).wait()

    return run(idx_t, table)


def kernel(x, embedding):
    n_samp, width = x.shape
    idx_t = jnp.swapaxes(x, 0, 1).astype(jnp.int32)
    out = _flat_gather(idx_t, embedding)
    return out.reshape(width, n_samp, _D).swapaxes(0, 1)


# speedup vs baseline: 1.0163x; 1.0025x over previous
import functools

import jax
import jax.numpy as jnp
from jax import lax
from jax.experimental import pallas as pl
from jax.experimental.pallas import tpu as pltpu
from jax.experimental.pallas import tpu_sc as plsc

_D = 128
_NC = 2
_NS = 16
_NW = _NC * _NS
_NB = 7


@jax.jit
def _flat_gather(idx_t, table):
    nch, n_samp = idx_t.shape
    cb = n_samp // _NW
    mesh = plsc.VectorSubcoreMesh(core_axis_name="c", subcore_axis_name="s")

    @functools.partial(
        pl.kernel,
        out_type=jax.ShapeDtypeStruct((nch * n_samp, _D), jnp.float32),
        mesh=mesh,
        scratch_types=[
            pltpu.VMEM((nch, cb), jnp.int32),
            pltpu.VMEM((_NB, cb, _D), jnp.float32),
            pltpu.SemaphoreType.DMA,
            pltpu.SemaphoreType.DMA,
        ],
    )
    def run(idx_hbm, table_hbm, out_hbm, idx_v, rows_v, gsem, osem):
        wid = lax.axis_index("s") * _NC + lax.axis_index("c")
        col0 = wid * cb
        pltpu.sync_copy(idx_hbm.at[:, pl.ds(col0, cb)], idx_v)

        def g_copy(ch, b):
            return pltpu.make_async_copy(
                table_hbm.at[idx_v.at[ch]], rows_v.at[b], gsem
            )

        def o_copy(ch, b):
            return pltpu.make_async_copy(
                rows_v.at[b], out_hbm.at[pl.ds(ch * n_samp + col0, cb)], osem
            )

        g_copy(0, 0).start()
        g_copy(1, 1).start()

        @pl.loop(0, nch)
        def chunk_loop(ch):
            b = lax.rem(ch, _NB)
            g_copy(ch, b).wait()
            o_copy(ch, b).start()

            @pl.when(ch + 2 < nch)
            def _():
                @pl.when(ch >= _NB - 2)
                def _():
                    o_copy(ch - _NB + 2, lax.rem(ch + 2, _NB)).wait()

                g_copy(ch + 2, lax.rem(ch + 2, _NB)).start()

        @pl.loop(nch - _NB, nch)
        def drain(k):
            o_copy(k, lax.rem(k, _NB)).wait()

    return run(idx_t, table)


def kernel(x, embedding):
    n_samp, width = x.shape
    idx_t = jnp.swapaxes(x, 0, 1).astype(jnp.int32)
    out = _flat_gather(idx_t, embedding)
    return out.reshape(width, n_samp, _D).swapaxes(0, 1)


# speedup vs baseline: 1.0264x; 1.0099x over previous
import functools

import jax
import jax.numpy as jnp
from jax import lax
from jax.experimental import pallas as pl
from jax.experimental.pallas import tpu as pltpu
from jax.experimental.pallas import tpu_sc as plsc

_D = 128
_NC = 2
_NS = 16
_NW = _NC * _NS
_NB = 7


@jax.jit
def _flat_gather(idx_t, table):
    nch, n_samp = idx_t.shape
    cb = n_samp // _NW
    mesh = plsc.VectorSubcoreMesh(core_axis_name="c", subcore_axis_name="s")

    @functools.partial(
        pl.kernel,
        out_type=jax.ShapeDtypeStruct((nch * n_samp, _D), jnp.float32),
        mesh=mesh,
        scratch_types=[
            pltpu.VMEM((nch, cb), jnp.int32),
            pltpu.VMEM((_NB, cb, _D), jnp.float32),
            pltpu.SemaphoreType.DMA,
            pltpu.SemaphoreType.DMA,
        ],
    )
    def run(idx_hbm, table_hbm, out_hbm, idx_v, rows_v, gsem, osem):
        wid = lax.axis_index("s") * _NC + lax.axis_index("c")
        col0 = wid * cb
        pltpu.sync_copy(idx_hbm.at[:, pl.ds(col0, cb)], idx_v)

        def g_copy(ch, b):
            return pltpu.make_async_copy(
                table_hbm.at[idx_v.at[ch]], rows_v.at[b], gsem
            )

        def o_copy(ch, b):
            return pltpu.make_async_copy(
                rows_v.at[b], out_hbm.at[pl.ds(ch * n_samp + col0, cb)], osem
            )

        g_copy(0, 0).start()
        g_copy(1, 1).start()
        g_copy(2, 2).start()

        @pl.loop(0, nch)
        def chunk_loop(ch):
            b = lax.rem(ch, _NB)
            g_copy(ch, b).wait()
            o_copy(ch, b).start()

            @pl.when(ch + 3 < nch)
            def _():
                @pl.when(ch >= _NB - 3)
                def _():
                    o_copy(ch - _NB + 3, lax.rem(ch + 3, _NB)).wait()

                g_copy(ch + 3, lax.rem(ch + 3, _NB)).start()

        @pl.loop(nch - _NB, nch)
        def drain(k):
            o_copy(k, lax.rem(k, _NB)).wait()

    return run(idx_t, table)


def kernel(x, embedding):
    n_samp, width = x.shape
    idx_t = jnp.swapaxes(x, 0, 1).astype(jnp.int32)
    out = _flat_gather(idx_t, embedding)
    return out.reshape(width, n_samp, _D).swapaxes(0, 1)
